# baseline (speedup 1.0000x reference)
import functools

import jax
import jax.numpy as jnp
import numpy as np
from jax import lax
from jax.experimental import pallas as pl
from jax.experimental.pallas import tpu as pltpu
from jax.experimental.pallas import tpu_sc as plsc

D = 128
LANES = 16
CHUNK = 128
NBUF = 4
NBLK = D // LANES
WPR = D // 2

_GDN = lax.GatherDimensionNumbers(
    offset_dims=(), collapsed_slice_dims=(0,), start_index_map=(0,))


def _lane_shuffle(x, idx):
    return lax.gather(x, idx[:, None], _GDN, (1,),
                      mode=lax.GatherScatterMode.PROMISE_IN_BOUNDS)


def _tree_reduce(vs):
    iota = jnp.arange(LANES, dtype=jnp.int32)
    s = 1
    while len(vs) > 1:
        mask = (iota & s) == 0
        perm = iota ^ s
        nxt = []
        for i in range(0, len(vs), 2):
            a, b = vs[i], vs[i + 1]
            keep = jnp.where(mask, a, b)
            other = _lane_shuffle(jnp.where(mask, b, a), perm)
            nxt.append(keep + other)
        vs = nxt
        s *= 2
    return vs[0]


def _sc_body(num_cores):
    def body(idx_hbm, ent_hbm, tblw_hbm, out_hbm,
             idx, erows, tblw, outv, sem_tbl, *sems):
        iota = jnp.arange(LANES, dtype=jnp.int32)
        offs = [iota + LANES * b2 for b2 in range(WPR // LANES)]
        splats = [jnp.full((LANES,), r, jnp.int32) for r in range(LANES)]
        wid = lax.axis_index("s") * num_cores + lax.axis_index("c")
        nch = idx.shape[1]
        pltpu.sync_copy(idx_hbm.at[wid], idx)
        tbl_h = pltpu.async_copy(tblw_hbm, tblw, sem_tbl)

        def start_e(j):
            return pltpu.async_copy(
                ent_hbm.at[idx.at[0, j]], erows.at[j % NBUF], sems[j % NBUF])

        eh = {}
        for j in range(min(NBUF, nch)):
            eh[j] = start_e(j)
        tbl_h.wait()

        for j in range(nch):
            eh.pop(j).wait()
            ebuf = erows.at[j % NBUF]

            def group_body(g, _, ebuf=ebuf, j=j, offs=offs, splats=splats):
                base = g * LANES
                ti = idx[1, j, pl.ds(base, LANES)]
                ws = ti * WPR
                vs = []
                for r in range(LANES):
                    wsr = _lane_shuffle(ws, splats[r])
                    tb = []
                    for b2 in range(WPR // LANES):
                        w = plsc.load_gather(tblw, [wsr + offs[b2]])
                        pair = plsc.unpack(plsc.bitcast(w, jnp.bfloat16),
                                           format=plsc.PackFormat.INTERLEAVED)
                        tb.extend(pair)
                    prods = []
                    for b in range(NBLK):
                        e = ebuf[base + r, pl.ds(b * LANES, LANES)]
                        prods.append(e * tb[b])
                    while len(prods) > 1:
                        prods = [prods[i] + prods[i + 1]
                                 for i in range(0, len(prods), 2)]
                    vs.append(prods[0])
                outv[pl.ds(base, LANES)] = _tree_reduce(vs)
                return 0

            lax.fori_loop(0, CHUNK // LANES, group_body, 0)
            if j + NBUF < nch:
                eh[j + NBUF] = start_e(j + NBUF)
            pltpu.sync_copy(outv, out_hbm.at[wid, pl.ds(j * CHUNK, CHUNK)])

    return body


def _pack_type_table(type_embedding):
    nt = type_embedding.shape[0]
    perm = np.empty((D,), np.int32)
    for p in range(D // 32):
        for i in range(16):
            perm[32 * p + 2 * i] = 32 * p + i
            perm[32 * p + 2 * i + 1] = 32 * p + 16 + i
    shuf = type_embedding.astype(jnp.bfloat16)[:, perm]
    return lax.bitcast_convert_type(
        shuf.reshape(nt, WPR, 2), jnp.int32).reshape(nt * WPR)


def kernel(entity, pos_type, ent_emb, type_embedding):
    B = entity.shape[0]
    info = plsc.get_sparse_core_info()
    nw = info.num_cores * info.num_subcores
    bpw = B // nw
    nch = bpw // CHUNK
    mesh = plsc.VectorSubcoreMesh(core_axis_name="c", subcore_axis_name="s")
    idx = jnp.stack(
        [entity.astype(jnp.int32).reshape(nw, nch, CHUNK),
         pos_type.astype(jnp.int32).reshape(nw, nch, CHUNK)], axis=1)
    tblw = _pack_type_table(type_embedding)
    k = functools.partial(
        pl.kernel,
        mesh=mesh,
        compiler_params=pltpu.CompilerParams(
            needs_layout_passes=False,
            disable_bounds_checks=True,
            disable_semaphore_checks=True),
        out_type=jax.ShapeDtypeStruct((nw, bpw), jnp.float32),
        scratch_types=[
            pltpu.VMEM((2, nch, CHUNK), jnp.int32),
            pltpu.VMEM((NBUF, CHUNK, D), jnp.float32),
            pltpu.VMEM((tblw.shape[0],), jnp.int32),
            pltpu.VMEM((CHUNK,), jnp.float32),
            pltpu.SemaphoreType.DMA,
        ] + [pltpu.SemaphoreType.DMA] * NBUF,
    )(_sc_body(info.num_cores))
    out = k(idx, ent_emb, tblw)
    return out.reshape(B, 1)


# speedup vs baseline: 1.0269x; 1.0269x over previous
import functools

import jax
import jax.numpy as jnp
import numpy as np
from jax import lax
from jax.experimental import pallas as pl
from jax.experimental.pallas import tpu as pltpu
from jax.experimental.pallas import tpu_sc as plsc

D = 128
LANES = 16
CHUNK = 128
NBUF = 3
NBLK = D // LANES
WPR = D // 2

_GDN = lax.GatherDimensionNumbers(
    offset_dims=(), collapsed_slice_dims=(0,), start_index_map=(0,))


def _lane_shuffle(x, idx):
    return lax.gather(x, idx[:, None], _GDN, (1,),
                      mode=lax.GatherScatterMode.PROMISE_IN_BOUNDS)


def _tree_reduce(vs):
    iota = jnp.arange(LANES, dtype=jnp.int32)
    s = 1
    while len(vs) > 1:
        mask = (iota & s) == 0
        perm = iota ^ s
        nxt = []
        for i in range(0, len(vs), 2):
            a, b = vs[i], vs[i + 1]
            keep = jnp.where(mask, a, b)
            other = _lane_shuffle(jnp.where(mask, b, a), perm)
            nxt.append(keep + other)
        vs = nxt
        s *= 2
    return vs[0]


def _sc_body(num_cores):
    def body(idx_hbm, ent_hbm, tblw_hbm, out_hbm,
             idx, erows, tblw, outv, sem_tbl, sem_out, *sems):
        iota = jnp.arange(LANES, dtype=jnp.int32)
        offs = [iota + LANES * b2 for b2 in range(WPR // LANES)]
        splats = [jnp.full((LANES,), r, jnp.int32) for r in range(LANES)]
        wid = lax.axis_index("s") * num_cores + lax.axis_index("c")
        nch = idx.shape[1]
        pltpu.sync_copy(idx_hbm.at[wid], idx)
        tbl_h = pltpu.async_copy(tblw_hbm, tblw, sem_tbl)

        def start_e(j):
            return pltpu.async_copy(
                ent_hbm.at[idx.at[0, j]], erows.at[j % NBUF], sems[j % NBUF])

        eh = {}
        for j in range(min(NBUF, nch)):
            eh[j] = start_e(j)
        tbl_h.wait()

        oh = []
        for j in range(nch):
            eh.pop(j).wait()
            ebuf = erows.at[j % NBUF]

            def group_body(g, _, ebuf=ebuf, j=j, offs=offs, splats=splats):
                base = g * LANES
                ti = idx[1, j, pl.ds(base, LANES)]
                ws = ti * WPR
                vs = []
                for r in range(LANES):
                    wsr = _lane_shuffle(ws, splats[r])
                    tb = []
                    for b2 in range(WPR // LANES):
                        w = plsc.load_gather(tblw, [wsr + offs[b2]])
                        pair = plsc.unpack(plsc.bitcast(w, jnp.bfloat16),
                                           format=plsc.PackFormat.INTERLEAVED)
                        tb.extend(pair)
                    prods = []
                    for b in range(NBLK):
                        e = ebuf[base + r, pl.ds(b * LANES, LANES)]
                        prods.append(e * tb[b])
                    while len(prods) > 1:
                        prods = [prods[i] + prods[i + 1]
                                 for i in range(0, len(prods), 2)]
                    vs.append(prods[0])
                outv[j, pl.ds(base, LANES)] = _tree_reduce(vs)
                return 0

            lax.fori_loop(0, CHUNK // LANES, group_body, 0)
            if j + NBUF < nch:
                eh[j + NBUF] = start_e(j + NBUF)
            oh.append(pltpu.async_copy(
                outv.at[j], out_hbm.at[wid, pl.ds(j * CHUNK, CHUNK)], sem_out))
        for h in oh:
            h.wait()

    return body


def _pack_type_table(type_embedding):
    nt = type_embedding.shape[0]
    perm = np.empty((D,), np.int32)
    for p in range(D // 32):
        for i in range(16):
            perm[32 * p + 2 * i] = 32 * p + i
            perm[32 * p + 2 * i + 1] = 32 * p + 16 + i
    shuf = type_embedding.astype(jnp.bfloat16)[:, perm]
    return lax.bitcast_convert_type(
        shuf.reshape(nt, WPR, 2), jnp.int32).reshape(nt * WPR)


def kernel(entity, pos_type, ent_emb, type_embedding):
    B = entity.shape[0]
    info = plsc.get_sparse_core_info()
    nw = info.num_cores * info.num_subcores
    bpw = B // nw
    nch = bpw // CHUNK
    mesh = plsc.VectorSubcoreMesh(core_axis_name="c", subcore_axis_name="s")
    idx = jnp.stack(
        [entity.astype(jnp.int32).reshape(nw, nch, CHUNK),
         pos_type.astype(jnp.int32).reshape(nw, nch, CHUNK)], axis=1)
    tblw = _pack_type_table(type_embedding)
    k = functools.partial(
        pl.kernel,
        mesh=mesh,
        compiler_params=pltpu.CompilerParams(needs_layout_passes=False),
        out_type=jax.ShapeDtypeStruct((nw, bpw), jnp.float32),
        scratch_types=[
            pltpu.VMEM((2, nch, CHUNK), jnp.int32),
            pltpu.VMEM((NBUF, CHUNK, D), jnp.float32),
            pltpu.VMEM((tblw.shape[0],), jnp.int32),
            pltpu.VMEM((nch, CHUNK), jnp.float32),
            pltpu.SemaphoreType.DMA,
            pltpu.SemaphoreType.DMA,
        ] + [pltpu.SemaphoreType.DMA] * NBUF,
    )(_sc_body(info.num_cores))
    out = k(idx, ent_emb, tblw)
    return out.reshape(B, 1)
